# baseline (speedup 1.0000x reference)
import functools

import jax
import jax.numpy as jnp
from jax.experimental import pallas as pl
from jax.experimental.pallas import tpu as pltpu

M = 4096
N = 4096
BM = 2048
BN = 512


def _cumsum_kernel(x_ref, mask_ref, out_ref, carry_ref):
    j = pl.program_id(1)

    @pl.when(j == 0)
    def _():
        carry_ref[...] = jnp.zeros_like(carry_ref)

    masked = jnp.where(mask_ref[...] != 0, x_ref[...], 0.0).astype(jnp.bfloat16)

    rows = jax.lax.broadcasted_iota(jnp.int32, (BN, BN), 0)
    cols = jax.lax.broadcasted_iota(jnp.int32, (BN, BN), 1)
    tri = (rows <= cols).astype(jnp.bfloat16)

    csum = jax.lax.dot(masked, tri, preferred_element_type=jnp.float32)

    carry = carry_ref[:, :1]
    out_ref[...] = csum + carry
    carry_ref[...] = jnp.broadcast_to(carry + csum[:, -1:], carry_ref.shape)


@jax.jit
def kernel(x, mask):
    mask = mask.astype(jnp.int8)
    grid = (M // BM, N // BN)
    return pl.pallas_call(
        _cumsum_kernel,
        grid=grid,
        in_specs=[
            pl.BlockSpec((BM, BN), lambda i, j: (i, j)),
            pl.BlockSpec((BM, BN), lambda i, j: (i, j)),
        ],
        out_specs=pl.BlockSpec((BM, BN), lambda i, j: (i, j)),
        out_shape=jax.ShapeDtypeStruct((M, N), jnp.float32),
        scratch_shapes=[pltpu.VMEM((BM, 128), jnp.float32)],
        compiler_params=pltpu.CompilerParams(
            dimension_semantics=("parallel", "arbitrary"),
        ),
    )(x, mask)


# speedup vs baseline: 1.1147x; 1.1147x over previous
import functools

import jax
import jax.numpy as jnp
from jax.experimental import pallas as pl
from jax.experimental.pallas import tpu as pltpu

M = 4096
N = 4096
BM = 4096
BN = 512


def _cumsum_kernel(x_ref, mask_ref, out_ref, carry_ref):
    j = pl.program_id(1)

    @pl.when(j == 0)
    def _():
        carry_ref[...] = jnp.zeros_like(carry_ref)

    masked = x_ref[...].astype(jnp.bfloat16) * mask_ref[...].astype(jnp.bfloat16)

    rows = jax.lax.broadcasted_iota(jnp.int32, (BN, BN), 0)
    cols = jax.lax.broadcasted_iota(jnp.int32, (BN, BN), 1)
    tri = (rows <= cols).astype(jnp.bfloat16)

    csum = jax.lax.dot(masked, tri, preferred_element_type=jnp.float32)

    carry = carry_ref[:, :1]
    out_ref[...] = csum + carry
    carry_ref[...] = jnp.broadcast_to(carry + csum[:, -1:], carry_ref.shape)


@jax.jit
def kernel(x, mask):
    mask = mask.astype(jnp.int4)
    grid = (M // BM, N // BN)
    return pl.pallas_call(
        _cumsum_kernel,
        grid=grid,
        in_specs=[
            pl.BlockSpec((BM, BN), lambda i, j: (i, j)),
            pl.BlockSpec((BM, BN), lambda i, j: (i, j)),
        ],
        out_specs=pl.BlockSpec((BM, BN), lambda i, j: (i, j)),
        out_shape=jax.ShapeDtypeStruct((M, N), jnp.float32),
        scratch_shapes=[pltpu.VMEM((BM, 128), jnp.float32)],
        compiler_params=pltpu.CompilerParams(
            dimension_semantics=("arbitrary", "arbitrary"),
        ),
    )(x, mask)
